# Initial kernel scaffold; baseline (speedup 1.0000x reference)
import jax
import jax.numpy as jnp
from jax.experimental import pallas as pl


def kernel(x, edge_index, W1, b1, W2, b2):
    raise NotImplementedError("write your pallas kernel here")


# speedup vs baseline: 18.4472x; 18.4472x over previous
import functools

import jax
import jax.numpy as jnp
from jax import lax
from jax.experimental import pallas as pl
from jax.experimental.pallas import tpu as pltpu
from jax.experimental.pallas import tpu_sc as plsc

CH = 80
FS = 64
NCORE = 2
NSUB = 16
NW = NCORE * NSUB


def _mesh():
    return plsc.VectorSubcoreMesh(core_axis_name="c", subcore_axis_name="s")


def _make_deg(N, E):
    nch = E // NW // CH

    @functools.partial(
        pl.kernel,
        out_type=jax.ShapeDtypeStruct((NCORE, N), jnp.float32),
        mesh=_mesh(),
        scratch_types=[
            pltpu.VMEM((nch, CH), jnp.int32),
            pltpu.VMEM((CH,), jnp.float32),
            pltpu.VMEM_SHARED((N,), jnp.float32),
        ],
    )
    def deg_kernel(dst_hbm, zeros_hbm, out_hbm, idx_v, ones_v, acc):
        c = lax.axis_index("c")
        s = lax.axis_index("s")
        wid = c * NSUB + s
        for j in range(CH // 16):
            ones_v[pl.ds(j * 16, 16)] = jnp.ones((16,), jnp.float32)

        @pl.when(s == 0)
        def _():
            pltpu.sync_copy(zeros_hbm, acc)

        pltpu.sync_copy(dst_hbm.at[wid], idx_v)
        plsc.subcore_barrier()

        def body(i, carry):
            pltpu.sync_copy(ones_v, acc.at[idx_v.at[i]], add=True)
            return carry

        lax.fori_loop(0, nch, body, 0)
        plsc.subcore_barrier()

        @pl.when(s == 0)
        def _():
            pltpu.sync_copy(acc, out_hbm.at[c])

    return deg_kernel


def _make_agg(N, E, npass):
    nch = E // NSUB // CH

    @functools.partial(
        pl.kernel,
        out_type=jax.ShapeDtypeStruct((NCORE * npass, N, FS), jnp.float32),
        mesh=_mesh(),
        compiler_params=pltpu.CompilerParams(use_tc_tiling_on_sc=False),
        scratch_types=[
            pltpu.VMEM((npass, nch, CH), jnp.int32),
            pltpu.VMEM((nch, CH), jnp.int32),
            pltpu.VMEM((CH, FS), jnp.float32),
            pltpu.VMEM((CH, FS), jnp.float32),
            pltpu.VMEM_SHARED((N, FS), jnp.float32),
            pltpu.SemaphoreType.DMA,
            pltpu.SemaphoreType.DMA,
        ],
    )
    def agg_kernel(table_hbm, gidx_hbm, didx_hbm, zeros_hbm, out_hbm,
                   gidx_v, didx_v, bufa, bufb, acc, sema, semb):
        c = lax.axis_index("c")
        s = lax.axis_index("s")
        wid = c * NSUB + s
        pltpu.sync_copy(didx_hbm.at[wid], didx_v)
        pltpu.sync_copy(gidx_hbm.at[wid], gidx_v)

        for p in range(npass):
            @pl.when(s == 0)
            def _():
                pltpu.sync_copy(zeros_hbm, acc)

            plsc.subcore_barrier()
            gv = gidx_v.at[p]

            def start(i, buf, sem):
                pltpu.make_async_copy(
                    table_hbm.at[gv.at[i]], buf, sem).start()

            def finish(i, buf, sem):
                pltpu.make_async_copy(
                    table_hbm.at[gv.at[i]], buf, sem).wait()

            def scat(i, buf):
                pltpu.sync_copy(buf, acc.at[didx_v.at[i]], add=True)

            start(0, bufa, sema)

            def body(i2, carry):
                i = i2 * 2
                start(i + 1, bufb, semb)
                finish(i, bufa, sema)
                scat(i, bufa)

                @pl.when(i + 2 < nch)
                def _():
                    start(i + 2, bufa, sema)

                finish(i + 1, bufb, semb)
                scat(i + 1, bufb)
                return carry

            lax.fori_loop(0, nch // 2, body, 0)
            plsc.subcore_barrier()

            @pl.when(s == 0)
            def _():
                pltpu.sync_copy(acc, out_hbm.at[c * npass + p])

            plsc.subcore_barrier()

    return agg_kernel


def _tc1(x, degp, W1, bm):
    N, D = x.shape
    H = W1.shape[1]
    ns = H // FS
    grid = (N // bm,)

    def body(x_ref, d_ref, w_ref, g_ref, dinv_ref):
        deg = d_ref[0] + d_ref[1] + 1.0
        dinv = lax.rsqrt(deg)
        g = jnp.dot(x_ref[...] * dinv, w_ref[...],
                    preferred_element_type=jnp.float32)
        for q in range(ns):
            g_ref[q] = g[:, q * FS:(q + 1) * FS]
        dinv_ref[...] = dinv

    return pl.pallas_call(
        body,
        grid=grid,
        in_specs=[
            pl.BlockSpec((bm, D), lambda i: (i, 0)),
            pl.BlockSpec((2, bm, 1), lambda i: (0, i, 0)),
            pl.BlockSpec((D, H), lambda i: (0, 0)),
        ],
        out_specs=[
            pl.BlockSpec((ns, bm, FS), lambda i: (0, i, 0)),
            pl.BlockSpec((bm, 1), lambda i: (i, 0)),
        ],
        out_shape=[
            jax.ShapeDtypeStruct((ns, N, FS), jnp.float32),
            jax.ShapeDtypeStruct((N, 1), jnp.float32),
        ],
    )(x, degp, W1)


def _tc2(S1, g1, dinv, b1, W2, bm):
    N = dinv.shape[0]
    H2, DO = W2.shape
    ns_in = H2 // FS
    ns_out = DO // FS
    grid = (N // bm,)

    def body(s_ref, g_ref, dinv_ref, b_ref, w_ref, out_ref):
        dv = dinv_ref[...]
        b = b_ref[...]
        w = w_ref[...]
        g2 = None
        for q in range(ns_in):
            h = jnp.maximum(
                dv * (s_ref[q] + g_ref[q]) + b[:, q * FS:(q + 1) * FS], 0.0)
            d = jnp.dot(dv * h, w[q * FS:(q + 1) * FS],
                        preferred_element_type=jnp.float32)
            g2 = d if g2 is None else g2 + d
        for q in range(ns_out):
            out_ref[q] = g2[:, q * FS:(q + 1) * FS]

    return pl.pallas_call(
        body,
        grid=grid,
        in_specs=[
            pl.BlockSpec((ns_in, bm, FS), lambda i: (0, i, 0)),
            pl.BlockSpec((ns_in, bm, FS), lambda i: (0, i, 0)),
            pl.BlockSpec((bm, 1), lambda i: (i, 0)),
            pl.BlockSpec((1, H2), lambda i: (0, 0)),
            pl.BlockSpec((H2, DO), lambda i: (0, 0)),
        ],
        out_specs=pl.BlockSpec((ns_out, bm, FS), lambda i: (0, i, 0)),
        out_shape=jax.ShapeDtypeStruct((ns_out, N, FS), jnp.float32),
    )(S1, g1, dinv, b1, W2)


def _tc3(S2, g2, dinv, b2, bm):
    DO = b2.shape[1]
    ns = DO // FS
    N = dinv.shape[0]
    grid = (N // bm,)

    def body(s_ref, g_ref, dinv_ref, b_ref, out_ref):
        dv = dinv_ref[...]
        b = b_ref[...]
        for q in range(ns):
            out_ref[:, q * FS:(q + 1) * FS] = jnp.maximum(
                dv * (s_ref[q] + g_ref[q]) + b[:, q * FS:(q + 1) * FS], 0.0)

    return pl.pallas_call(
        body,
        grid=grid,
        in_specs=[
            pl.BlockSpec((ns, bm, FS), lambda i: (0, i, 0)),
            pl.BlockSpec((ns, bm, FS), lambda i: (0, i, 0)),
            pl.BlockSpec((bm, 1), lambda i: (i, 0)),
            pl.BlockSpec((1, DO), lambda i: (0, 0)),
        ],
        out_specs=pl.BlockSpec((bm, DO), lambda i: (i, 0)),
        out_shape=jax.ShapeDtypeStruct((N, DO), jnp.float32),
    )(S2, g2, dinv, b2)


def kernel(x, edge_index, W1, b1, W2, b2):
    N, D = x.shape
    E = edge_index.shape[1]
    H = W1.shape[1]
    DO = W2.shape[1]
    assert E % (NW * CH) == 0 and N % NSUB == 0
    assert H == 4 * FS and DO == 2 * FS

    src = edge_index[0]
    dst = edge_index[1]
    zeros1 = jnp.zeros((N,), jnp.float32)
    zeros2 = jnp.zeros((N, FS), jnp.float32)
    bm = 1000

    nch_w = E // NW // CH
    nch_s = E // NSUB // CH
    src16 = src.reshape(NSUB, nch_s, CH)
    dst16 = dst.reshape(NSUB, nch_s, CH)
    didx = jnp.concatenate([dst16, dst16], axis=0)

    degp = _make_deg(N, E)(dst.reshape(NW, nch_w, CH), zeros1)

    g1, dinv = _tc1(x, degp.reshape(NCORE, N, 1), W1, bm)

    offs1 = (jnp.arange(NCORE * 2, dtype=src.dtype) * N).reshape(
        NCORE, 2, 1, 1)
    gidx1 = (src16[:, None] + offs1[:, None]).reshape(NW, 2, nch_s, CH)
    S1 = _make_agg(N, E, 2)(g1.reshape(4 * N, FS), gidx1, didx, zeros2)

    g2 = _tc2(S1, g1, dinv, b1.reshape(1, H), W2, bm)

    offs2 = (jnp.arange(NCORE, dtype=src.dtype) * N).reshape(NCORE, 1, 1, 1)
    gidx2 = (src16[:, None] + offs2[:, None]).reshape(NW, 1, nch_s, CH)
    S2 = _make_agg(N, E, 1)(g2.reshape(2 * N, FS), gidx2, didx, zeros2)

    return _tc3(S2, g2, dinv, b2.reshape(1, DO), bm)


# speedup vs baseline: 26.1621x; 1.4182x over previous
import functools

import jax
import jax.numpy as jnp
from jax import lax
from jax.experimental import pallas as pl
from jax.experimental.pallas import tpu as pltpu
from jax.experimental.pallas import tpu_sc as plsc

CH = 80
FS = 64
NCORE = 2
NSUB = 16
NW = NCORE * NSUB


def _mesh():
    return plsc.VectorSubcoreMesh(core_axis_name="c", subcore_axis_name="s")


def _make_deg(N, E):
    nch = E // NW // CH

    @functools.partial(
        pl.kernel,
        out_type=jax.ShapeDtypeStruct((NCORE, N), jnp.float32),
        mesh=_mesh(),
        scratch_types=[
            pltpu.VMEM((nch, CH), jnp.int32),
            pltpu.VMEM((CH,), jnp.float32),
            pltpu.VMEM_SHARED((N,), jnp.float32),
        ],
    )
    def deg_kernel(dst_hbm, zeros_hbm, out_hbm, idx_v, ones_v, acc):
        c = lax.axis_index("c")
        s = lax.axis_index("s")
        wid = c * NSUB + s
        for j in range(CH // 16):
            ones_v[pl.ds(j * 16, 16)] = jnp.ones((16,), jnp.float32)

        @pl.when(s == 0)
        def _():
            pltpu.sync_copy(zeros_hbm, acc)

        pltpu.sync_copy(dst_hbm.at[wid], idx_v)
        plsc.subcore_barrier()

        def body(i, carry):
            pltpu.sync_copy(ones_v, acc.at[idx_v.at[i]], add=True)
            return carry

        lax.fori_loop(0, nch, body, 0)
        plsc.subcore_barrier()

        @pl.when(s == 0)
        def _():
            pltpu.sync_copy(acc, out_hbm.at[c])

    return deg_kernel


def _make_agg(N, E, npass):
    nch = E // NSUB // CH

    @functools.partial(
        pl.kernel,
        out_type=jax.ShapeDtypeStruct((NCORE * npass, N, FS), jnp.float32),
        mesh=_mesh(),
        compiler_params=pltpu.CompilerParams(use_tc_tiling_on_sc=False),
        scratch_types=[
            pltpu.VMEM((npass, nch, CH), jnp.int32),
            pltpu.VMEM((nch, CH), jnp.int32),
            pltpu.VMEM((CH, FS), jnp.float32),
            pltpu.VMEM((CH, FS), jnp.float32),
            pltpu.VMEM_SHARED((N, FS), jnp.float32),
            pltpu.SemaphoreType.DMA,
            pltpu.SemaphoreType.DMA,
        ],
    )
    def agg_kernel(table_hbm, gidx_hbm, didx_hbm, zeros_hbm, out_hbm,
                   gidx_v, didx_v, bufa, bufb, acc, sema, semb):
        c = lax.axis_index("c")
        s = lax.axis_index("s")
        wid = c * NSUB + s
        pltpu.sync_copy(didx_hbm.at[wid], didx_v)
        pltpu.sync_copy(gidx_hbm.at[wid], gidx_v)

        for p in range(npass):
            @pl.when(s == 0)
            def _():
                pltpu.sync_copy(zeros_hbm, acc)

            plsc.subcore_barrier()
            gv = gidx_v.at[p]

            def start(i, buf, sem):
                pltpu.make_async_copy(
                    table_hbm.at[gv.at[i]], buf, sem).start()

            def finish(i, buf, sem):
                pltpu.make_async_copy(
                    table_hbm.at[gv.at[i]], buf, sem).wait()

            def scat(i, buf):
                pltpu.sync_copy(buf, acc.at[didx_v.at[i]], add=True)

            start(0, bufa, sema)

            def body(i2, carry):
                i = i2 * 2
                start(i + 1, bufb, semb)
                finish(i, bufa, sema)
                scat(i, bufa)

                @pl.when(i + 2 < nch)
                def _():
                    start(i + 2, bufa, sema)

                finish(i + 1, bufb, semb)
                scat(i + 1, bufb)
                return carry

            lax.fori_loop(0, nch // 2, body, 0)
            plsc.subcore_barrier()

            @pl.when(s == 0)
            def _():
                pltpu.sync_copy(acc, out_hbm.at[c * npass + p])

            plsc.subcore_barrier()

    return agg_kernel


def _tc1(x, degp, bm):
    N, D = x.shape
    ns = D // FS
    grid = (N // bm,)

    def body(x_ref, d_ref, y_ref, dinv_ref):
        deg = d_ref[0] + d_ref[1] + 1.0
        dinv = lax.rsqrt(deg)
        y = x_ref[...] * dinv
        for q in range(ns):
            y_ref[q] = y[:, q * FS:(q + 1) * FS]
        dinv_ref[...] = dinv

    return pl.pallas_call(
        body,
        grid=grid,
        in_specs=[
            pl.BlockSpec((bm, D), lambda i: (i, 0)),
            pl.BlockSpec((2, bm, 1), lambda i: (0, i, 0)),
        ],
        out_specs=[
            pl.BlockSpec((ns, bm, FS), lambda i: (0, i, 0)),
            pl.BlockSpec((bm, 1), lambda i: (i, 0)),
        ],
        out_shape=[
            jax.ShapeDtypeStruct((ns, N, FS), jnp.float32),
            jax.ShapeDtypeStruct((N, 1), jnp.float32),
        ],
    )(x, degp)


def _tc2(Sx, y, dinv, b1, W1, W2, bm):
    N = dinv.shape[0]
    D, H = W1.shape
    DO = W2.shape[1]
    ns_in = D // FS
    ns_out = DO // FS
    grid = (N // bm,)

    def body(s_ref, y_ref, dinv_ref, b_ref, w1_ref, w2_ref, out_ref):
        dv = dinv_ref[...]
        a = jnp.concatenate(
            [s_ref[q] + y_ref[q] for q in range(ns_in)], axis=1)
        z = jnp.dot(a, w1_ref[...], preferred_element_type=jnp.float32)
        h = jnp.maximum(dv * z + b_ref[...], 0.0)
        g2 = jnp.dot(dv * h, w2_ref[...],
                     preferred_element_type=jnp.float32)
        for q in range(ns_out):
            out_ref[q] = g2[:, q * FS:(q + 1) * FS]

    return pl.pallas_call(
        body,
        grid=grid,
        in_specs=[
            pl.BlockSpec((ns_in, bm, FS), lambda i: (0, i, 0)),
            pl.BlockSpec((ns_in, bm, FS), lambda i: (0, i, 0)),
            pl.BlockSpec((bm, 1), lambda i: (i, 0)),
            pl.BlockSpec((1, H), lambda i: (0, 0)),
            pl.BlockSpec((D, H), lambda i: (0, 0)),
            pl.BlockSpec((H, DO), lambda i: (0, 0)),
        ],
        out_specs=pl.BlockSpec((ns_out, bm, FS), lambda i: (0, i, 0)),
        out_shape=jax.ShapeDtypeStruct((ns_out, N, FS), jnp.float32),
    )(Sx, y, dinv, b1, W1, W2)


def _tc3(S2, g2, dinv, b2, bm):
    DO = b2.shape[1]
    ns = DO // FS
    N = dinv.shape[0]
    grid = (N // bm,)

    def body(s_ref, g_ref, dinv_ref, b_ref, out_ref):
        dv = dinv_ref[...]
        b = b_ref[...]
        for q in range(ns):
            out_ref[:, q * FS:(q + 1) * FS] = jnp.maximum(
                dv * (s_ref[q] + g_ref[q]) + b[:, q * FS:(q + 1) * FS], 0.0)

    return pl.pallas_call(
        body,
        grid=grid,
        in_specs=[
            pl.BlockSpec((ns, bm, FS), lambda i: (0, i, 0)),
            pl.BlockSpec((ns, bm, FS), lambda i: (0, i, 0)),
            pl.BlockSpec((bm, 1), lambda i: (i, 0)),
            pl.BlockSpec((1, DO), lambda i: (0, 0)),
        ],
        out_specs=pl.BlockSpec((bm, DO), lambda i: (i, 0)),
        out_shape=jax.ShapeDtypeStruct((N, DO), jnp.float32),
    )(S2, g2, dinv, b2)


def kernel(x, edge_index, W1, b1, W2, b2):
    N, D = x.shape
    E = edge_index.shape[1]
    H = W1.shape[1]
    DO = W2.shape[1]
    assert E % (NW * CH) == 0 and N % NSUB == 0
    assert D == 2 * FS and DO == 2 * FS

    src = edge_index[0]
    dst = edge_index[1]
    zeros1 = jnp.zeros((N,), jnp.float32)
    zeros2 = jnp.zeros((N, FS), jnp.float32)
    bm = 1000

    nch_w = E // NW // CH
    nch_s = E // NSUB // CH
    src16 = src.reshape(NSUB, nch_s, CH)
    dst16 = dst.reshape(NSUB, nch_s, CH)
    didx = jnp.concatenate([dst16, dst16], axis=0)

    offs = (jnp.arange(NCORE, dtype=src.dtype) * N).reshape(NCORE, 1, 1, 1)
    gidx = (src16[:, None] + offs[:, None]).reshape(NW, 1, nch_s, CH)
    agg = _make_agg(N, E, 1)

    degp = _make_deg(N, E)(dst.reshape(NW, nch_w, CH), zeros1)

    y, dinv = _tc1(x, degp.reshape(NCORE, N, 1), bm)

    Sx = agg(y.reshape(2 * N, FS), gidx, didx, zeros2)

    g2 = _tc2(Sx, y, dinv, b1.reshape(1, H), W1, W2, bm)

    S2 = agg(g2.reshape(2 * N, FS), gidx, didx, zeros2)

    return _tc3(S2, g2, dinv, b2.reshape(1, DO), bm)


# speedup vs baseline: 35.5660x; 1.3594x over previous
import functools

import jax
import jax.numpy as jnp
from jax import lax
from jax.experimental import pallas as pl
from jax.experimental.pallas import tpu as pltpu
from jax.experimental.pallas import tpu_sc as plsc

CH = 80
FS = 64
NCORE = 2
NSUB = 16
NW = NCORE * NSUB


def _mesh():
    return plsc.VectorSubcoreMesh(core_axis_name="c", subcore_axis_name="s")


def _make_deg(N, E):
    nch = E // NW // CH

    @functools.partial(
        pl.kernel,
        out_type=jax.ShapeDtypeStruct((NCORE, N), jnp.float32),
        mesh=_mesh(),
        scratch_types=[
            pltpu.VMEM((nch, CH), jnp.int32),
            pltpu.VMEM((CH,), jnp.float32),
            pltpu.VMEM_SHARED((N,), jnp.float32),
        ],
    )
    def deg_kernel(dst_hbm, zeros_hbm, out_hbm, idx_v, ones_v, acc):
        c = lax.axis_index("c")
        s = lax.axis_index("s")
        wid = c * NSUB + s
        for j in range(CH // 16):
            ones_v[pl.ds(j * 16, 16)] = jnp.ones((16,), jnp.float32)

        @pl.when(s == 0)
        def _():
            pltpu.sync_copy(zeros_hbm, acc)

        pltpu.sync_copy(dst_hbm.at[wid], idx_v)
        plsc.subcore_barrier()

        def body(i, carry):
            pltpu.sync_copy(ones_v, acc.at[idx_v.at[i]], add=True)
            return carry

        lax.fori_loop(0, nch, body, 0)
        plsc.subcore_barrier()

        @pl.when(s == 0)
        def _():
            pltpu.sync_copy(acc, out_hbm.at[c])

    return deg_kernel


def _make_agg(N, E, npass):
    nch = E // NSUB // CH
    nbuf = 4

    @functools.partial(
        pl.kernel,
        out_type=jax.ShapeDtypeStruct((NCORE * npass, N, FS), jnp.float32),
        mesh=_mesh(),
        compiler_params=pltpu.CompilerParams(use_tc_tiling_on_sc=False),
        scratch_types=[
            pltpu.VMEM((npass, nch, CH), jnp.int32),
            pltpu.VMEM((nch, CH), jnp.int32),
            pltpu.VMEM((nbuf, CH, FS), jnp.float32),
            pltpu.VMEM_SHARED((N, FS), jnp.float32),
            pltpu.SemaphoreType.DMA((nbuf,)),
        ],
    )
    def agg_kernel(table_hbm, gidx_hbm, didx_hbm, zeros_hbm, out_hbm,
                   gidx_v, didx_v, bufs, acc, sems):
        c = lax.axis_index("c")
        s = lax.axis_index("s")
        wid = c * NSUB + s
        pltpu.sync_copy(didx_hbm.at[wid], didx_v)
        pltpu.sync_copy(gidx_hbm.at[wid], gidx_v)

        for p in range(npass):
            @pl.when(s == 0)
            def _():
                pltpu.sync_copy(zeros_hbm, acc)

            plsc.subcore_barrier()
            gv = gidx_v.at[p]

            def start(i, k):
                pltpu.make_async_copy(
                    table_hbm.at[gv.at[i]], bufs.at[k], sems.at[k]).start()

            def finish(i, k):
                pltpu.make_async_copy(
                    table_hbm.at[gv.at[i]], bufs.at[k], sems.at[k]).wait()

            def scat(i, k):
                pltpu.sync_copy(bufs.at[k], acc.at[didx_v.at[i]], add=True)

            for j in range(nbuf - 1):
                start(j, j)

            def body(i, carry):
                k = lax.rem(i, nbuf)

                @pl.when(i + nbuf - 1 < nch)
                def _():
                    start(i + nbuf - 1, lax.rem(i + nbuf - 1, nbuf))

                finish(i, k)
                scat(i, k)
                return carry

            lax.fori_loop(0, nch, body, 0)
            plsc.subcore_barrier()

            @pl.when(s == 0)
            def _():
                pltpu.sync_copy(acc, out_hbm.at[c * npass + p])

            plsc.subcore_barrier()

    return agg_kernel


def _tc1(x, degp, bm):
    N, D = x.shape
    ns = D // FS
    grid = (N // bm,)

    def body(x_ref, d_ref, y_ref, dinv_ref):
        deg = d_ref[0] + d_ref[1] + 1.0
        dinv = lax.rsqrt(deg)
        y = x_ref[...] * dinv
        for q in range(ns):
            y_ref[q] = y[:, q * FS:(q + 1) * FS]
        dinv_ref[...] = dinv

    return pl.pallas_call(
        body,
        grid=grid,
        in_specs=[
            pl.BlockSpec((bm, D), lambda i: (i, 0)),
            pl.BlockSpec((2, bm, 1), lambda i: (0, i, 0)),
        ],
        out_specs=[
            pl.BlockSpec((ns, bm, FS), lambda i: (0, i, 0)),
            pl.BlockSpec((bm, 1), lambda i: (i, 0)),
        ],
        out_shape=[
            jax.ShapeDtypeStruct((ns, N, FS), jnp.float32),
            jax.ShapeDtypeStruct((N, 1), jnp.float32),
        ],
    )(x, degp)


def _tc2(Sx, y, dinv, b1, W1, W2, bm):
    N = dinv.shape[0]
    D, H = W1.shape
    DO = W2.shape[1]
    ns_in = D // FS
    ns_out = DO // FS
    grid = (N // bm,)

    def body(s_ref, y_ref, dinv_ref, b_ref, w1_ref, w2_ref, out_ref):
        dv = dinv_ref[...]
        a = jnp.concatenate(
            [s_ref[q] + y_ref[q] for q in range(ns_in)], axis=1)
        z = jnp.dot(a, w1_ref[...], preferred_element_type=jnp.float32)
        h = jnp.maximum(dv * z + b_ref[...], 0.0)
        g2 = jnp.dot(dv * h, w2_ref[...],
                     preferred_element_type=jnp.float32)
        for q in range(ns_out):
            out_ref[q] = g2[:, q * FS:(q + 1) * FS]

    return pl.pallas_call(
        body,
        grid=grid,
        in_specs=[
            pl.BlockSpec((ns_in, bm, FS), lambda i: (0, i, 0)),
            pl.BlockSpec((ns_in, bm, FS), lambda i: (0, i, 0)),
            pl.BlockSpec((bm, 1), lambda i: (i, 0)),
            pl.BlockSpec((1, H), lambda i: (0, 0)),
            pl.BlockSpec((D, H), lambda i: (0, 0)),
            pl.BlockSpec((H, DO), lambda i: (0, 0)),
        ],
        out_specs=pl.BlockSpec((ns_out, bm, FS), lambda i: (0, i, 0)),
        out_shape=jax.ShapeDtypeStruct((ns_out, N, FS), jnp.float32),
    )(Sx, y, dinv, b1, W1, W2)


def _tc3(S2, g2, dinv, b2, bm):
    DO = b2.shape[1]
    ns = DO // FS
    N = dinv.shape[0]
    grid = (N // bm,)

    def body(s_ref, g_ref, dinv_ref, b_ref, out_ref):
        dv = dinv_ref[...]
        b = b_ref[...]
        for q in range(ns):
            out_ref[:, q * FS:(q + 1) * FS] = jnp.maximum(
                dv * (s_ref[q] + g_ref[q]) + b[:, q * FS:(q + 1) * FS], 0.0)

    return pl.pallas_call(
        body,
        grid=grid,
        in_specs=[
            pl.BlockSpec((ns, bm, FS), lambda i: (0, i, 0)),
            pl.BlockSpec((ns, bm, FS), lambda i: (0, i, 0)),
            pl.BlockSpec((bm, 1), lambda i: (i, 0)),
            pl.BlockSpec((1, DO), lambda i: (0, 0)),
        ],
        out_specs=pl.BlockSpec((bm, DO), lambda i: (i, 0)),
        out_shape=jax.ShapeDtypeStruct((N, DO), jnp.float32),
    )(S2, g2, dinv, b2)


def kernel(x, edge_index, W1, b1, W2, b2):
    N, D = x.shape
    E = edge_index.shape[1]
    H = W1.shape[1]
    DO = W2.shape[1]
    assert E % (NW * CH) == 0 and N % NSUB == 0
    assert D == 2 * FS and DO == 2 * FS

    src = edge_index[0]
    dst = edge_index[1]
    zeros1 = jnp.zeros((N,), jnp.float32)
    zeros2 = jnp.zeros((N, FS), jnp.float32)
    bm = 1000

    nch_w = E // NW // CH
    nch_s = E // NSUB // CH
    src16 = src.reshape(NSUB, nch_s, CH)
    dst16 = dst.reshape(NSUB, nch_s, CH)
    didx = jnp.concatenate([dst16, dst16], axis=0)

    offs = (jnp.arange(NCORE, dtype=src.dtype) * N).reshape(NCORE, 1, 1, 1)
    gidx = (src16[:, None] + offs[:, None]).reshape(NW, 1, nch_s, CH)
    agg = _make_agg(N, E, 1)

    degp = _make_deg(N, E)(dst.reshape(NW, nch_w, CH), zeros1)

    y, dinv = _tc1(x, degp.reshape(NCORE, N, 1), bm)

    Sx = agg(y.reshape(2 * N, FS), gidx, didx, zeros2)

    g2 = _tc2(Sx, y, dinv, b1.reshape(1, H), W1, W2, bm)

    S2 = agg(g2.reshape(2 * N, FS), gidx, didx, zeros2)

    return _tc3(S2, g2, dinv, b2.reshape(1, DO), bm)


# speedup vs baseline: 36.1584x; 1.0167x over previous
import functools

import jax
import jax.numpy as jnp
from jax import lax
from jax.experimental import pallas as pl
from jax.experimental.pallas import tpu as pltpu
from jax.experimental.pallas import tpu_sc as plsc

CH = 80
FS = 64
NCORE = 2
NSUB = 16
NW = NCORE * NSUB


def _mesh():
    return plsc.VectorSubcoreMesh(core_axis_name="c", subcore_axis_name="s")


def _make_deg(N, E):
    nch = E // NW // CH

    @functools.partial(
        pl.kernel,
        out_type=jax.ShapeDtypeStruct((NCORE, N), jnp.float32),
        mesh=_mesh(),
        scratch_types=[
            pltpu.VMEM((nch, CH), jnp.int32),
            pltpu.VMEM((CH,), jnp.float32),
            pltpu.VMEM_SHARED((N,), jnp.float32),
        ],
    )
    def deg_kernel(dst_hbm, zeros_hbm, out_hbm, idx_v, ones_v, acc):
        c = lax.axis_index("c")
        s = lax.axis_index("s")
        wid = c * NSUB + s
        for j in range(CH // 16):
            ones_v[pl.ds(j * 16, 16)] = jnp.ones((16,), jnp.float32)

        @pl.when(s == 0)
        def _():
            pltpu.sync_copy(zeros_hbm, acc)

        pltpu.sync_copy(dst_hbm.at[wid], idx_v)
        plsc.subcore_barrier()

        def body(i, carry):
            pltpu.sync_copy(ones_v, acc.at[idx_v.at[i]], add=True)
            return carry

        lax.fori_loop(0, nch, body, 0)
        plsc.subcore_barrier()

        @pl.when(s == 0)
        def _():
            pltpu.sync_copy(acc, out_hbm.at[c])

    return deg_kernel


def _make_agg(N, E, npass):
    nch = E // NSUB // CH
    nbuf = 8

    @functools.partial(
        pl.kernel,
        out_type=jax.ShapeDtypeStruct((NCORE * npass, N, FS), jnp.float32),
        mesh=_mesh(),
        compiler_params=pltpu.CompilerParams(use_tc_tiling_on_sc=False),
        scratch_types=[
            pltpu.VMEM((npass, nch, CH), jnp.int32),
            pltpu.VMEM((nch, CH), jnp.int32),
            pltpu.VMEM((nbuf, CH, FS), jnp.float32),
            pltpu.VMEM_SHARED((N, FS), jnp.float32),
            pltpu.SemaphoreType.DMA((nbuf,)),
        ],
    )
    def agg_kernel(table_hbm, gidx_hbm, didx_hbm, zeros_hbm, out_hbm,
                   gidx_v, didx_v, bufs, acc, sems):
        c = lax.axis_index("c")
        s = lax.axis_index("s")
        wid = c * NSUB + s
        pltpu.sync_copy(didx_hbm.at[wid], didx_v)
        pltpu.sync_copy(gidx_hbm.at[wid], gidx_v)

        for p in range(npass):
            @pl.when(s == 0)
            def _():
                pltpu.sync_copy(zeros_hbm, acc)

            plsc.subcore_barrier()
            gv = gidx_v.at[p]

            def start(i, k):
                pltpu.make_async_copy(
                    table_hbm.at[gv.at[i]], bufs.at[k], sems.at[k]).start()

            def finish(i, k):
                pltpu.make_async_copy(
                    table_hbm.at[gv.at[i]], bufs.at[k], sems.at[k]).wait()

            def scat(i, k):
                pltpu.sync_copy(bufs.at[k], acc.at[didx_v.at[i]], add=True)

            for j in range(nbuf - 1):
                start(j, j)

            def body(i, carry):
                k = lax.rem(i, nbuf)

                @pl.when(i + nbuf - 1 < nch)
                def _():
                    start(i + nbuf - 1, lax.rem(i + nbuf - 1, nbuf))

                finish(i, k)
                scat(i, k)
                return carry

            lax.fori_loop(0, nch, body, 0)
            plsc.subcore_barrier()

            @pl.when(s == 0)
            def _():
                pltpu.sync_copy(acc, out_hbm.at[c * npass + p])

            plsc.subcore_barrier()

    return agg_kernel


def _tc1(x, degp, bm):
    N, D = x.shape
    ns = D // FS
    grid = (N // bm,)

    def body(x_ref, d_ref, y_ref, dinv_ref):
        deg = d_ref[0] + d_ref[1] + 1.0
        dinv = lax.rsqrt(deg)
        y = x_ref[...] * dinv
        for q in range(ns):
            y_ref[q] = y[:, q * FS:(q + 1) * FS]
        dinv_ref[...] = dinv

    return pl.pallas_call(
        body,
        grid=grid,
        in_specs=[
            pl.BlockSpec((bm, D), lambda i: (i, 0)),
            pl.BlockSpec((2, bm, 1), lambda i: (0, i, 0)),
        ],
        out_specs=[
            pl.BlockSpec((ns, bm, FS), lambda i: (0, i, 0)),
            pl.BlockSpec((bm, 1), lambda i: (i, 0)),
        ],
        out_shape=[
            jax.ShapeDtypeStruct((ns, N, FS), jnp.float32),
            jax.ShapeDtypeStruct((N, 1), jnp.float32),
        ],
    )(x, degp)


def _tc2(Sx, y, dinv, b1, W1, W2, bm):
    N = dinv.shape[0]
    D, H = W1.shape
    DO = W2.shape[1]
    ns_in = D // FS
    ns_out = DO // FS
    grid = (N // bm,)

    def body(s_ref, y_ref, dinv_ref, b_ref, w1_ref, w2_ref, out_ref):
        dv = dinv_ref[...]
        a = jnp.concatenate(
            [s_ref[q] + y_ref[q] for q in range(ns_in)], axis=1)
        z = jnp.dot(a, w1_ref[...], preferred_element_type=jnp.float32)
        h = jnp.maximum(dv * z + b_ref[...], 0.0)
        g2 = jnp.dot(dv * h, w2_ref[...],
                     preferred_element_type=jnp.float32)
        for q in range(ns_out):
            out_ref[q] = g2[:, q * FS:(q + 1) * FS]

    return pl.pallas_call(
        body,
        grid=grid,
        in_specs=[
            pl.BlockSpec((ns_in, bm, FS), lambda i: (0, i, 0)),
            pl.BlockSpec((ns_in, bm, FS), lambda i: (0, i, 0)),
            pl.BlockSpec((bm, 1), lambda i: (i, 0)),
            pl.BlockSpec((1, H), lambda i: (0, 0)),
            pl.BlockSpec((D, H), lambda i: (0, 0)),
            pl.BlockSpec((H, DO), lambda i: (0, 0)),
        ],
        out_specs=pl.BlockSpec((ns_out, bm, FS), lambda i: (0, i, 0)),
        out_shape=jax.ShapeDtypeStruct((ns_out, N, FS), jnp.float32),
    )(Sx, y, dinv, b1, W1, W2)


def _tc3(S2, g2, dinv, b2, bm):
    DO = b2.shape[1]
    ns = DO // FS
    N = dinv.shape[0]
    grid = (N // bm,)

    def body(s_ref, g_ref, dinv_ref, b_ref, out_ref):
        dv = dinv_ref[...]
        b = b_ref[...]
        for q in range(ns):
            out_ref[:, q * FS:(q + 1) * FS] = jnp.maximum(
                dv * (s_ref[q] + g_ref[q]) + b[:, q * FS:(q + 1) * FS], 0.0)

    return pl.pallas_call(
        body,
        grid=grid,
        in_specs=[
            pl.BlockSpec((ns, bm, FS), lambda i: (0, i, 0)),
            pl.BlockSpec((ns, bm, FS), lambda i: (0, i, 0)),
            pl.BlockSpec((bm, 1), lambda i: (i, 0)),
            pl.BlockSpec((1, DO), lambda i: (0, 0)),
        ],
        out_specs=pl.BlockSpec((bm, DO), lambda i: (i, 0)),
        out_shape=jax.ShapeDtypeStruct((N, DO), jnp.float32),
    )(S2, g2, dinv, b2)


def kernel(x, edge_index, W1, b1, W2, b2):
    N, D = x.shape
    E = edge_index.shape[1]
    H = W1.shape[1]
    DO = W2.shape[1]
    assert E % (NW * CH) == 0 and N % NSUB == 0
    assert D == 2 * FS and DO == 2 * FS

    src = edge_index[0]
    dst = edge_index[1]
    zeros1 = jnp.zeros((N,), jnp.float32)
    zeros2 = jnp.zeros((N, FS), jnp.float32)
    bm = 1000

    nch_w = E // NW // CH
    nch_s = E // NSUB // CH
    src16 = src.reshape(NSUB, nch_s, CH)
    dst16 = dst.reshape(NSUB, nch_s, CH)
    didx = jnp.concatenate([dst16, dst16], axis=0)

    offs = (jnp.arange(NCORE, dtype=src.dtype) * N).reshape(NCORE, 1, 1, 1)
    gidx = (src16[:, None] + offs[:, None]).reshape(NW, 1, nch_s, CH)
    agg = _make_agg(N, E, 1)

    degp = _make_deg(N, E)(dst.reshape(NW, nch_w, CH), zeros1)

    y, dinv = _tc1(x, degp.reshape(NCORE, N, 1), bm)

    Sx = agg(y.reshape(2 * N, FS), gidx, didx, zeros2)

    g2 = _tc2(Sx, y, dinv, b1.reshape(1, H), W1, W2, bm)

    S2 = agg(g2.reshape(2 * N, FS), gidx, didx, zeros2)

    return _tc3(S2, g2, dinv, b2.reshape(1, DO), bm)


# speedup vs baseline: 36.2064x; 1.0013x over previous
import functools

import jax
import jax.numpy as jnp
from jax import lax
from jax.experimental import pallas as pl
from jax.experimental.pallas import tpu as pltpu
from jax.experimental.pallas import tpu_sc as plsc

CH = 80
FS = 64
NCORE = 2
NSUB = 16
NW = NCORE * NSUB


def _mesh():
    return plsc.VectorSubcoreMesh(core_axis_name="c", subcore_axis_name="s")


def _make_deg(N, E):
    nch = E // NW // CH

    @functools.partial(
        pl.kernel,
        out_type=jax.ShapeDtypeStruct((NCORE, N), jnp.float32),
        mesh=_mesh(),
        scratch_types=[
            pltpu.VMEM((nch, CH), jnp.int32),
            pltpu.VMEM((CH,), jnp.float32),
            pltpu.VMEM_SHARED((N,), jnp.float32),
        ],
    )
    def deg_kernel(dst_hbm, zeros_hbm, out_hbm, idx_v, ones_v, acc):
        c = lax.axis_index("c")
        s = lax.axis_index("s")
        wid = c * NSUB + s
        for j in range(CH // 16):
            ones_v[pl.ds(j * 16, 16)] = jnp.ones((16,), jnp.float32)

        @pl.when(s == 0)
        def _():
            pltpu.sync_copy(zeros_hbm, acc)

        pltpu.sync_copy(dst_hbm.at[wid], idx_v)
        plsc.subcore_barrier()

        def body(i, carry):
            pltpu.sync_copy(ones_v, acc.at[idx_v.at[i]], add=True)
            return carry

        lax.fori_loop(0, nch, body, 0)
        plsc.subcore_barrier()

        @pl.when(s == 0)
        def _():
            pltpu.sync_copy(acc, out_hbm.at[c])

    return deg_kernel


def _make_agg(N, E, npass):
    nch = E // NSUB // CH
    nbuf = 8

    @functools.partial(
        pl.kernel,
        out_type=jax.ShapeDtypeStruct((NCORE * npass, N, FS), jnp.float32),
        mesh=_mesh(),
        compiler_params=pltpu.CompilerParams(use_tc_tiling_on_sc=False),
        scratch_types=[
            pltpu.VMEM((npass, nch, CH), jnp.int32),
            pltpu.VMEM((nch, CH), jnp.int32),
            pltpu.VMEM((nbuf, CH, FS), jnp.float32),
            pltpu.VMEM_SHARED((N, FS), jnp.float32),
            pltpu.SemaphoreType.DMA((nbuf,)),
            pltpu.SemaphoreType.DMA((nbuf,)),
        ],
    )
    def agg_kernel(table_hbm, gidx_hbm, didx_hbm, zeros_hbm, out_hbm,
                   gidx_v, didx_v, bufs, acc, gsems, ssems):
        c = lax.axis_index("c")
        s = lax.axis_index("s")
        wid = c * NSUB + s
        pltpu.sync_copy(didx_hbm.at[wid], didx_v)
        pltpu.sync_copy(gidx_hbm.at[wid], gidx_v)

        for p in range(npass):
            @pl.when(s == 0)
            def _():
                pltpu.sync_copy(zeros_hbm, acc)

            plsc.subcore_barrier()
            gv = gidx_v.at[p]

            def gat(i, k):
                return pltpu.make_async_copy(
                    table_hbm.at[gv.at[i]], bufs.at[k], gsems.at[k])

            def scat(i, k):
                return pltpu.make_async_copy(
                    bufs.at[k], acc.at[didx_v.at[i]], ssems.at[k])

            for j in range(nbuf - 1):
                gat(j, j).start()

            def body(i, carry):
                k = lax.rem(i, nbuf)
                gat(i, k).wait()
                scat(i, k).start(add=True)
                k2 = lax.rem(i + nbuf - 1, nbuf)

                @pl.when(i >= 1)
                def _():
                    scat(i - 1, k2).wait()

                @pl.when(i + nbuf - 1 < nch)
                def _():
                    gat(i + nbuf - 1, k2).start()

                return carry

            lax.fori_loop(0, nch, body, 0)
            scat(nch - 1, lax.rem(nch - 1, nbuf)).wait()
            plsc.subcore_barrier()

            @pl.when(s == 0)
            def _():
                pltpu.sync_copy(acc, out_hbm.at[c * npass + p])

            plsc.subcore_barrier()

    return agg_kernel


def _tc1(x, degp, bm):
    N, D = x.shape
    ns = D // FS
    grid = (N // bm,)

    def body(x_ref, d_ref, y_ref, dinv_ref):
        deg = d_ref[0] + d_ref[1] + 1.0
        dinv = lax.rsqrt(deg)
        y = x_ref[...] * dinv
        for q in range(ns):
            y_ref[q] = y[:, q * FS:(q + 1) * FS]
        dinv_ref[...] = dinv

    return pl.pallas_call(
        body,
        grid=grid,
        in_specs=[
            pl.BlockSpec((bm, D), lambda i: (i, 0)),
            pl.BlockSpec((2, bm, 1), lambda i: (0, i, 0)),
        ],
        out_specs=[
            pl.BlockSpec((ns, bm, FS), lambda i: (0, i, 0)),
            pl.BlockSpec((bm, 1), lambda i: (i, 0)),
        ],
        out_shape=[
            jax.ShapeDtypeStruct((ns, N, FS), jnp.float32),
            jax.ShapeDtypeStruct((N, 1), jnp.float32),
        ],
    )(x, degp)


def _tc2(Sx, y, dinv, b1, W1, W2, bm):
    N = dinv.shape[0]
    D, H = W1.shape
    DO = W2.shape[1]
    ns_in = D // FS
    ns_out = DO // FS
    grid = (N // bm,)

    def body(s_ref, y_ref, dinv_ref, b_ref, w1_ref, w2_ref, out_ref):
        dv = dinv_ref[...]
        a = jnp.concatenate(
            [s_ref[q] + y_ref[q] for q in range(ns_in)], axis=1)
        z = jnp.dot(a, w1_ref[...], preferred_element_type=jnp.float32)
        h = jnp.maximum(dv * z + b_ref[...], 0.0)
        g2 = jnp.dot(dv * h, w2_ref[...],
                     preferred_element_type=jnp.float32)
        for q in range(ns_out):
            out_ref[q] = g2[:, q * FS:(q + 1) * FS]

    return pl.pallas_call(
        body,
        grid=grid,
        in_specs=[
            pl.BlockSpec((ns_in, bm, FS), lambda i: (0, i, 0)),
            pl.BlockSpec((ns_in, bm, FS), lambda i: (0, i, 0)),
            pl.BlockSpec((bm, 1), lambda i: (i, 0)),
            pl.BlockSpec((1, H), lambda i: (0, 0)),
            pl.BlockSpec((D, H), lambda i: (0, 0)),
            pl.BlockSpec((H, DO), lambda i: (0, 0)),
        ],
        out_specs=pl.BlockSpec((ns_out, bm, FS), lambda i: (0, i, 0)),
        out_shape=jax.ShapeDtypeStruct((ns_out, N, FS), jnp.float32),
    )(Sx, y, dinv, b1, W1, W2)


def _tc3(S2, g2, dinv, b2, bm):
    DO = b2.shape[1]
    ns = DO // FS
    N = dinv.shape[0]
    grid = (N // bm,)

    def body(s_ref, g_ref, dinv_ref, b_ref, out_ref):
        dv = dinv_ref[...]
        b = b_ref[...]
        for q in range(ns):
            out_ref[:, q * FS:(q + 1) * FS] = jnp.maximum(
                dv * (s_ref[q] + g_ref[q]) + b[:, q * FS:(q + 1) * FS], 0.0)

    return pl.pallas_call(
        body,
        grid=grid,
        in_specs=[
            pl.BlockSpec((ns, bm, FS), lambda i: (0, i, 0)),
            pl.BlockSpec((ns, bm, FS), lambda i: (0, i, 0)),
            pl.BlockSpec((bm, 1), lambda i: (i, 0)),
            pl.BlockSpec((1, DO), lambda i: (0, 0)),
        ],
        out_specs=pl.BlockSpec((bm, DO), lambda i: (i, 0)),
        out_shape=jax.ShapeDtypeStruct((N, DO), jnp.float32),
    )(S2, g2, dinv, b2)


def kernel(x, edge_index, W1, b1, W2, b2):
    N, D = x.shape
    E = edge_index.shape[1]
    H = W1.shape[1]
    DO = W2.shape[1]
    assert E % (NW * CH) == 0 and N % NSUB == 0
    assert D == 2 * FS and DO == 2 * FS

    src = edge_index[0]
    dst = edge_index[1]
    zeros1 = jnp.zeros((N,), jnp.float32)
    zeros2 = jnp.zeros((N, FS), jnp.float32)
    bm = 1000

    nch_w = E // NW // CH
    nch_s = E // NSUB // CH
    src16 = src.reshape(NSUB, nch_s, CH)
    dst16 = dst.reshape(NSUB, nch_s, CH)
    didx = jnp.concatenate([dst16, dst16], axis=0)

    offs = (jnp.arange(NCORE, dtype=src.dtype) * N).reshape(NCORE, 1, 1, 1)
    gidx = (src16[:, None] + offs[:, None]).reshape(NW, 1, nch_s, CH)
    agg = _make_agg(N, E, 1)

    degp = _make_deg(N, E)(dst.reshape(NW, nch_w, CH), zeros1)

    y, dinv = _tc1(x, degp.reshape(NCORE, N, 1), bm)

    Sx = agg(y.reshape(2 * N, FS), gidx, didx, zeros2)

    g2 = _tc2(Sx, y, dinv, b1.reshape(1, H), W1, W2, bm)

    S2 = agg(g2.reshape(2 * N, FS), gidx, didx, zeros2)

    return _tc3(S2, g2, dinv, b2.reshape(1, DO), bm)
